# baseline (speedup 1.0000x reference)
import functools

import jax
import jax.numpy as jnp
from jax import lax
from jax.experimental import pallas as pl
from jax.experimental.pallas import tpu as pltpu
from jax.experimental.pallas import tpu_sc as plsc

B, S, D = 4, 8192, 1024
BS = 2048
NC, NS = 2, 16
NW = NC * NS
S_PER_W = S // NW
CS = 32
N_CHUNK = S_PER_W // CS


def _add_kernel(x_ref, emb_ref, out_ref):
    out_ref[0] = x_ref[0] + emb_ref[...]


def _tc_add(x3, emb):
    grid = (S // BS, 3)
    return pl.pallas_call(
        _add_kernel,
        grid=grid,
        in_specs=[
            pl.BlockSpec((1, BS, D), lambda s, b: (b, s, 0)),
            pl.BlockSpec((BS, D), lambda s, b: (s, 0)),
        ],
        out_specs=pl.BlockSpec((1, BS, D), lambda s, b: (b, s, 0)),
        out_shape=jax.ShapeDtypeStruct((3, S, D), x3.dtype),
        compiler_params=pltpu.CompilerParams(
            dimension_semantics=("parallel", "parallel"),
        ),
    )(x3, emb)


def _sc_add(x1, emb):
    mesh = plsc.VectorSubcoreMesh(
        core_axis_name="c", subcore_axis_name="s", num_cores=NC, num_subcores=NS
    )

    @functools.partial(
        pl.kernel,
        out_type=jax.ShapeDtypeStruct((S, D), jnp.float32),
        mesh=mesh,
        scratch_types=[
            pltpu.VMEM((CS, D), jnp.float32),
            pltpu.VMEM((CS, D), jnp.float32),
            pltpu.VMEM((CS, D), jnp.float32),
            pltpu.SemaphoreType.DMA,
            pltpu.SemaphoreType.DMA,
            pltpu.SemaphoreType.DMA,
            pltpu.SemaphoreType.DMA,
        ],
    )
    def k(x_hbm, emb_hbm, out_hbm, xa, xb, emb_v, la, lb, sa, sb):
        wid = lax.axis_index("s") * NC + lax.axis_index("c")
        s_base = wid * S_PER_W
        bufs = (xa, xb)
        lsems = (la, lb)
        ssems = (sa, sb)

        def rows(c):
            return pl.ds(s_base + c * CS, CS)

        pltpu.async_copy(x_hbm.at[rows(0)], xa, la)

        for c in range(N_CHUNK):
            p = c % 2
            q = 1 - p
            pltpu.sync_copy(emb_hbm.at[rows(c)], emb_v)

            if c + 1 < N_CHUNK:
                if c >= 1:
                    pltpu.make_async_copy(
                        bufs[q], out_hbm.at[rows(c - 1)], ssems[q]
                    ).wait()
                pltpu.async_copy(x_hbm.at[rows(c + 1)], bufs[q], lsems[q])

            pltpu.make_async_copy(x_hbm.at[rows(c)], bufs[p], lsems[p]).wait()
            xp = bufs[p]

            @plsc.parallel_loop(0, CS, step=1, unroll=2)
            def _(r):
                for col in range(D // 16):
                    sl = pl.ds(col * 16, 16)
                    plsc.addupdate(xp.at[r, sl], emb_v[r, sl])

            pltpu.async_copy(bufs[p], out_hbm.at[rows(c)], ssems[p])

        pltpu.make_async_copy(xa, out_hbm.at[rows(N_CHUNK - 2)], sa).wait()
        pltpu.make_async_copy(xb, out_hbm.at[rows(N_CHUNK - 1)], sb).wait()

    return k(x1, emb)


def kernel(x, emb):
    e = emb[:S]
    out_tc = _tc_add(x[:3], e)
    out_sc = _sc_add(x[3], e)
    return jnp.concatenate([out_tc, out_sc[None]], axis=0)


# speedup vs baseline: 3.1271x; 3.1271x over previous
import jax
import jax.numpy as jnp
from jax.experimental import pallas as pl
from jax.experimental.pallas import tpu as pltpu

B, S, D = 4, 8192, 1024
BS = 2048


def _add_kernel(x_ref, emb_ref, out_ref):
    out_ref[0] = x_ref[0] + emb_ref[...]


def kernel(x, emb):
    grid = (S // BS, B)
    return pl.pallas_call(
        _add_kernel,
        grid=grid,
        in_specs=[
            pl.BlockSpec((1, BS, D), lambda s, b: (b, s, 0)),
            pl.BlockSpec((BS, D), lambda s, b: (s, 0)),
        ],
        out_specs=pl.BlockSpec((1, BS, D), lambda s, b: (b, s, 0)),
        out_shape=jax.ShapeDtypeStruct((B, S, D), x.dtype),
        compiler_params=pltpu.CompilerParams(
            dimension_semantics=("parallel", "arbitrary"),
        ),
    )(x, emb[:S])
